# baseline (speedup 1.0000x reference)
import functools

import jax
import jax.numpy as jnp
from jax import lax
from jax.experimental import pallas as pl
from jax.experimental.pallas import tpu as pltpu
from jax.experimental.pallas import tpu_sc as plsc

N = 10000
NP = 10240
D_IN = 128
D_HID = 256
D_OUT = 128
E = 320000
C = 128
NC = 2
NS = 16
NW = NC * NS
NB = 2
NBD = 4
G = NBD * (-(-E // (NW * C * NBD)))
EP = G * NW * C
ROWS_PER_TILE = NP // NS
CNT0 = 40
CNT1 = 2 * G - CNT0
CNT_MAX = max(CNT0, CNT1)

_mesh = plsc.VectorSubcoreMesh(core_axis_name="c", subcore_axis_name="s")


def _deg_body(dstm, out, didx_all, onev, zvec, acc, s0, s1, s2, s3):
    cid = lax.axis_index("c")
    sid = lax.axis_index("s")
    wid = sid * NC + cid
    sems = [s0, s1, s2, s3]
    for i in range(8):
        onev[pl.ds(i * 16, 16)] = jnp.full((16,), 1.0, jnp.float32)
    for i in range(ROWS_PER_TILE // 16):
        zvec[pl.ds(i * 16, 16)] = jnp.zeros((16,), jnp.float32)
    pltpu.sync_copy(dstm.at[pl.ds(wid * G, G)], didx_all)
    pltpu.sync_copy(zvec, acc.at[pl.ds(sid * ROWS_PER_TILE, ROWS_PER_TILE)])
    plsc.subcore_barrier()

    for b in range(NBD):
        pltpu.async_copy(onev, acc.at[didx_all.at[b]], sems[b], add=True)

    def step(i, carry):
        go = i * NBD
        for b in range(NBD):
            pltpu.make_async_copy(onev, acc.at[didx_all.at[go + b - NBD]],
                                  sems[b]).wait()
            pltpu.async_copy(onev, acc.at[didx_all.at[go + b]], sems[b],
                             add=True)
        return carry

    lax.fori_loop(1, G // NBD, step, 0)
    for b in range(NBD):
        pltpu.make_async_copy(onev, acc.at[didx_all.at[G - NBD + b]],
                              sems[b]).wait()
    plsc.subcore_barrier()
    pltpu.sync_copy(
        acc.at[pl.ds(sid * ROWS_PER_TILE, ROWS_PER_TILE)],
        out.at[cid, pl.ds(sid * ROWS_PER_TILE, ROWS_PER_TILE)],
    )


_deg_call = functools.partial(
    pl.kernel,
    out_type=jax.ShapeDtypeStruct((NC, NP), jnp.float32),
    mesh=_mesh,
    scratch_types=[
        pltpu.VMEM((G, C), jnp.int32),
        pltpu.VMEM((C,), jnp.float32),
        pltpu.VMEM((ROWS_PER_TILE,), jnp.float32),
        pltpu.VMEM_SHARED((NP,), jnp.float32),
    ] + [pltpu.SemaphoreType.DMA] * NBD,
)(_deg_body)


def _agg_body(table, srcm, dstm, zeros2d, out, sidx_all, d0, d1,
              r0, r1, acc, g0, g1, t0, t1, i0, i1):
    cid = lax.axis_index("c")
    sid = lax.axis_index("s")
    wid = sid * NC + cid
    rows = [r0, r1]
    didx = [d0, d1]
    sem_g = [g0, g1]
    sem_s = [t0, t1]
    sem_i = [i0, i1]

    base = jnp.where(cid == 0, sid * CNT0, NS * CNT0 + sid * CNT1)
    cnt = jnp.where(cid == 0, CNT0, CNT1)

    pltpu.sync_copy(srcm.at[pl.ds(base, CNT_MAX)], sidx_all)
    pltpu.sync_copy(zeros2d, rows[0])
    for k in range(ROWS_PER_TILE // C):
        pltpu.sync_copy(rows[0], acc.at[pl.ds(sid * ROWS_PER_TILE + k * C, C)])
    plsc.subcore_barrier()

    for b in range(NB):
        pltpu.async_copy(dstm.at[base + b], didx[b], sem_i[b])
        pltpu.async_copy(table.at[sidx_all.at[b]], rows[b], sem_g[b])

    def step(i, carry):
        go = i * NB
        for b in range(NB):
            g = go + b
            pltpu.make_async_copy(table.at[sidx_all.at[g]], rows[b],
                                  sem_g[b]).wait()
            pltpu.make_async_copy(dstm.at[base + g], didx[b],
                                  sem_i[b]).wait()
            pltpu.sync_copy(rows[b], acc.at[didx[b]], add=True)
            pltpu.async_copy(dstm.at[base + g + NB], didx[b], sem_i[b])
            pltpu.async_copy(table.at[sidx_all.at[g + NB]], rows[b], sem_g[b])
        return carry

    lax.fori_loop(0, cnt // NB - 1, step, 0)
    for b in range(NB):
        g = cnt - NB + b
        pltpu.make_async_copy(table.at[sidx_all.at[g]], rows[b],
                              sem_g[b]).wait()
        pltpu.make_async_copy(dstm.at[base + g], didx[b], sem_i[b]).wait()
        pltpu.sync_copy(rows[b], acc.at[didx[b]], add=True)
    plsc.subcore_barrier()
    pltpu.sync_copy(
        acc.at[pl.ds(sid * ROWS_PER_TILE, ROWS_PER_TILE)],
        out.at[cid, pl.ds(sid * ROWS_PER_TILE, ROWS_PER_TILE)],
    )


_agg_call = functools.partial(
    pl.kernel,
    out_type=jax.ShapeDtypeStruct((NC, NP, D_IN), jnp.float32),
    mesh=_mesh,
    scratch_types=[
        pltpu.VMEM((CNT_MAX, C), jnp.int32),
        pltpu.VMEM((C,), jnp.int32),
        pltpu.VMEM((C,), jnp.int32),
        pltpu.VMEM((C, D_IN), jnp.float32),
        pltpu.VMEM((C, D_IN), jnp.float32),
        pltpu.VMEM_SHARED((NP, D_IN), jnp.float32),
    ] + [pltpu.SemaphoreType.DMA] * (3 * NB),
)(_agg_body)


ROW_BLK = 512
_GRID = (NP // ROW_BLK,)


def _scale_body(d0, d1, x, s_out, xs_out):
    s = lax.rsqrt(d0[...] + d1[...] + 1.0)
    s_out[...] = s
    xs_out[...] = x[...] * s


_scale_call = pl.pallas_call(
    _scale_body,
    grid=_GRID,
    in_specs=[
        pl.BlockSpec((ROW_BLK, 1), lambda i: (i, 0)),
        pl.BlockSpec((ROW_BLK, 1), lambda i: (i, 0)),
        pl.BlockSpec((ROW_BLK, D_IN), lambda i: (i, 0)),
    ],
    out_specs=[
        pl.BlockSpec((ROW_BLK, 1), lambda i: (i, 0)),
        pl.BlockSpec((ROW_BLK, D_IN), lambda i: (i, 0)),
    ],
    out_shape=[
        jax.ShapeDtypeStruct((NP, 1), jnp.float32),
        jax.ShapeDtypeStruct((NP, D_IN), jnp.float32),
    ],
)


def _layer_body(a0, a1, xs, s, w1, b1, w2, gs_out):
    z = (a0[...] + a1[...] + xs[...]) * s[...]
    h = jnp.dot(z, w1[...], preferred_element_type=jnp.float32) + b1[...]
    h = jnp.maximum(h, 0.0)
    g = jnp.dot(h, w2[...], preferred_element_type=jnp.float32)
    gs_out[...] = g * s[...]


_layer_call = pl.pallas_call(
    _layer_body,
    grid=_GRID,
    in_specs=[
        pl.BlockSpec((ROW_BLK, D_IN), lambda i: (i, 0)),
        pl.BlockSpec((ROW_BLK, D_IN), lambda i: (i, 0)),
        pl.BlockSpec((ROW_BLK, D_IN), lambda i: (i, 0)),
        pl.BlockSpec((ROW_BLK, 1), lambda i: (i, 0)),
        pl.BlockSpec((D_IN, D_HID), lambda i: (0, 0)),
        pl.BlockSpec((1, D_HID), lambda i: (0, 0)),
        pl.BlockSpec((D_HID, D_OUT), lambda i: (0, 0)),
    ],
    out_specs=pl.BlockSpec((ROW_BLK, D_OUT), lambda i: (i, 0)),
    out_shape=jax.ShapeDtypeStruct((NP, D_OUT), jnp.float32),
)


def _final_body(a0, a1, gs, s, b2, o_out):
    o_out[...] = (a0[...] + a1[...] + gs[...]) * s[...] + b2[...]


_final_call = pl.pallas_call(
    _final_body,
    grid=_GRID,
    in_specs=[
        pl.BlockSpec((ROW_BLK, D_OUT), lambda i: (i, 0)),
        pl.BlockSpec((ROW_BLK, D_OUT), lambda i: (i, 0)),
        pl.BlockSpec((ROW_BLK, D_OUT), lambda i: (i, 0)),
        pl.BlockSpec((ROW_BLK, 1), lambda i: (i, 0)),
        pl.BlockSpec((1, D_OUT), lambda i: (0, 0)),
    ],
    out_specs=pl.BlockSpec((ROW_BLK, D_OUT), lambda i: (i, 0)),
    out_shape=jax.ShapeDtypeStruct((NP, D_OUT), jnp.float32),
)


def kernel(x, edge_index, W1, b1, W2, b2):
    ei = edge_index.astype(jnp.int32)
    pad_e = EP - E
    src = jnp.concatenate([ei[0], jnp.zeros((pad_e,), jnp.int32)])
    dst = jnp.concatenate([ei[1], jnp.full((pad_e,), N, jnp.int32)])
    srcm = src.reshape(EP // C, C)
    dstm = dst.reshape(EP // C, C)
    xp = jnp.pad(x, ((0, NP - N), (0, 0)))
    zeros2d = jnp.zeros((C, D_IN), jnp.float32)

    deg = _deg_call(dstm)
    s, xs = _scale_call(deg[0].reshape(NP, 1), deg[1].reshape(NP, 1), xp)
    agg1 = _agg_call(xs, srcm, dstm, zeros2d)
    gs = _layer_call(agg1[0], agg1[1], xs, s,
                     W1, b1.reshape(1, D_HID), W2)
    agg2 = _agg_call(gs, srcm, dstm, zeros2d)
    outp = _final_call(agg2[0], agg2[1], gs, s, b2.reshape(1, D_OUT))
    return outp[:N]


# speedup vs baseline: 1.0816x; 1.0816x over previous
import functools

import jax
import jax.numpy as jnp
from jax import lax
from jax.experimental import pallas as pl
from jax.experimental.pallas import tpu as pltpu
from jax.experimental.pallas import tpu_sc as plsc

N = 10000
NP = 10240
D_IN = 128
D_HID = 256
D_OUT = 128
E = 320000
C = 128
NC = 2
NS = 16
NW = NC * NS
NB = 2
NBD = 4
G = NBD * (-(-E // (NW * C * NBD)))
EP = G * NW * C
ROWS_PER_TILE = NP // NS
CNT0 = 112
CNT1 = 2 * G - CNT0
CNT_MAX = max(CNT0, CNT1)

_mesh = plsc.VectorSubcoreMesh(core_axis_name="c", subcore_axis_name="s")


def _deg_body(dstm, out, didx_all, onev, zvec, acc, s0, s1, s2, s3):
    cid = lax.axis_index("c")
    sid = lax.axis_index("s")
    wid = sid * NC + cid
    sems = [s0, s1, s2, s3]
    for i in range(8):
        onev[pl.ds(i * 16, 16)] = jnp.full((16,), 1.0, jnp.float32)
    for i in range(ROWS_PER_TILE // 16):
        zvec[pl.ds(i * 16, 16)] = jnp.zeros((16,), jnp.float32)
    pltpu.sync_copy(dstm.at[pl.ds(wid * G, G)], didx_all)
    pltpu.sync_copy(zvec, acc.at[pl.ds(sid * ROWS_PER_TILE, ROWS_PER_TILE)])
    plsc.subcore_barrier()

    for b in range(NBD):
        pltpu.async_copy(onev, acc.at[didx_all.at[b]], sems[b], add=True)

    def step(i, carry):
        go = i * NBD
        for b in range(NBD):
            pltpu.make_async_copy(onev, acc.at[didx_all.at[go + b - NBD]],
                                  sems[b]).wait()
            pltpu.async_copy(onev, acc.at[didx_all.at[go + b]], sems[b],
                             add=True)
        return carry

    lax.fori_loop(1, G // NBD, step, 0)
    for b in range(NBD):
        pltpu.make_async_copy(onev, acc.at[didx_all.at[G - NBD + b]],
                              sems[b]).wait()
    plsc.subcore_barrier()
    pltpu.sync_copy(
        acc.at[pl.ds(sid * ROWS_PER_TILE, ROWS_PER_TILE)],
        out.at[cid, pl.ds(sid * ROWS_PER_TILE, ROWS_PER_TILE)],
    )


_deg_call = functools.partial(
    pl.kernel,
    out_type=jax.ShapeDtypeStruct((NC, NP), jnp.float32),
    mesh=_mesh,
    scratch_types=[
        pltpu.VMEM((G, C), jnp.int32),
        pltpu.VMEM((C,), jnp.float32),
        pltpu.VMEM((ROWS_PER_TILE,), jnp.float32),
        pltpu.VMEM_SHARED((NP,), jnp.float32),
    ] + [pltpu.SemaphoreType.DMA] * NBD,
)(_deg_body)


def _agg_body(table, srcm, dstm, zeros2d, out, sidx_all, d0, d1,
              r0, r1, acc, g0, g1, t0, t1, i0, i1):
    cid = lax.axis_index("c")
    sid = lax.axis_index("s")
    wid = sid * NC + cid
    rows = [r0, r1]
    didx = [d0, d1]
    sem_g = [g0, g1]
    sem_s = [t0, t1]
    sem_i = [i0, i1]

    base = jnp.where(cid == 0, sid * CNT0, NS * CNT0 + sid * CNT1)
    cnt = jnp.where(cid == 0, CNT0, CNT1)

    pltpu.sync_copy(srcm.at[pl.ds(base, CNT_MAX)], sidx_all)
    pltpu.sync_copy(zeros2d, rows[0])
    for k in range(ROWS_PER_TILE // C):
        pltpu.sync_copy(rows[0], acc.at[pl.ds(sid * ROWS_PER_TILE + k * C, C)])
    plsc.subcore_barrier()

    for b in range(NB):
        pltpu.async_copy(dstm.at[base + b], didx[b], sem_i[b])
        pltpu.async_copy(table.at[sidx_all.at[b]], rows[b], sem_g[b])

    def step(i, carry):
        go = i * NB
        for b in range(NB):
            g = go + b
            pltpu.make_async_copy(table.at[sidx_all.at[g]], rows[b],
                                  sem_g[b]).wait()
            pltpu.make_async_copy(dstm.at[base + g], didx[b],
                                  sem_i[b]).wait()
            pltpu.sync_copy(rows[b], acc.at[didx[b]], add=True)
            pltpu.async_copy(dstm.at[base + g + NB], didx[b], sem_i[b])
            pltpu.async_copy(table.at[sidx_all.at[g + NB]], rows[b], sem_g[b])
        return carry

    lax.fori_loop(0, cnt // NB - 1, step, 0)
    for b in range(NB):
        g = cnt - NB + b
        pltpu.make_async_copy(table.at[sidx_all.at[g]], rows[b],
                              sem_g[b]).wait()
        pltpu.make_async_copy(dstm.at[base + g], didx[b], sem_i[b]).wait()
        pltpu.sync_copy(rows[b], acc.at[didx[b]], add=True)
    plsc.subcore_barrier()
    pltpu.sync_copy(
        acc.at[pl.ds(sid * ROWS_PER_TILE, ROWS_PER_TILE)],
        out.at[cid, pl.ds(sid * ROWS_PER_TILE, ROWS_PER_TILE)],
    )


_agg_call = functools.partial(
    pl.kernel,
    out_type=jax.ShapeDtypeStruct((NC, NP, D_IN), jnp.float32),
    mesh=_mesh,
    scratch_types=[
        pltpu.VMEM((CNT_MAX, C), jnp.int32),
        pltpu.VMEM((C,), jnp.int32),
        pltpu.VMEM((C,), jnp.int32),
        pltpu.VMEM((C, D_IN), jnp.float32),
        pltpu.VMEM((C, D_IN), jnp.float32),
        pltpu.VMEM_SHARED((NP, D_IN), jnp.float32),
    ] + [pltpu.SemaphoreType.DMA] * (3 * NB),
)(_agg_body)


ROW_BLK = 512
_GRID = (NP // ROW_BLK,)


def _scale_body(d0, d1, x, s_out, xs_out):
    s = lax.rsqrt(d0[...] + d1[...] + 1.0)
    s_out[...] = s
    xs_out[...] = x[...] * s


_scale_call = pl.pallas_call(
    _scale_body,
    grid=_GRID,
    in_specs=[
        pl.BlockSpec((ROW_BLK, 1), lambda i: (i, 0)),
        pl.BlockSpec((ROW_BLK, 1), lambda i: (i, 0)),
        pl.BlockSpec((ROW_BLK, D_IN), lambda i: (i, 0)),
    ],
    out_specs=[
        pl.BlockSpec((ROW_BLK, 1), lambda i: (i, 0)),
        pl.BlockSpec((ROW_BLK, D_IN), lambda i: (i, 0)),
    ],
    out_shape=[
        jax.ShapeDtypeStruct((NP, 1), jnp.float32),
        jax.ShapeDtypeStruct((NP, D_IN), jnp.float32),
    ],
)


def _layer_body(a0, a1, xs, s, w1, b1, w2, gs_out):
    z = (a0[...] + a1[...] + xs[...]) * s[...]
    h = jnp.dot(z, w1[...], preferred_element_type=jnp.float32) + b1[...]
    h = jnp.maximum(h, 0.0)
    g = jnp.dot(h, w2[...], preferred_element_type=jnp.float32)
    gs_out[...] = g * s[...]


_layer_call = pl.pallas_call(
    _layer_body,
    grid=_GRID,
    in_specs=[
        pl.BlockSpec((ROW_BLK, D_IN), lambda i: (i, 0)),
        pl.BlockSpec((ROW_BLK, D_IN), lambda i: (i, 0)),
        pl.BlockSpec((ROW_BLK, D_IN), lambda i: (i, 0)),
        pl.BlockSpec((ROW_BLK, 1), lambda i: (i, 0)),
        pl.BlockSpec((D_IN, D_HID), lambda i: (0, 0)),
        pl.BlockSpec((1, D_HID), lambda i: (0, 0)),
        pl.BlockSpec((D_HID, D_OUT), lambda i: (0, 0)),
    ],
    out_specs=pl.BlockSpec((ROW_BLK, D_OUT), lambda i: (i, 0)),
    out_shape=jax.ShapeDtypeStruct((NP, D_OUT), jnp.float32),
)


def _final_body(a0, a1, gs, s, b2, o_out):
    o_out[...] = (a0[...] + a1[...] + gs[...]) * s[...] + b2[...]


_final_call = pl.pallas_call(
    _final_body,
    grid=_GRID,
    in_specs=[
        pl.BlockSpec((ROW_BLK, D_OUT), lambda i: (i, 0)),
        pl.BlockSpec((ROW_BLK, D_OUT), lambda i: (i, 0)),
        pl.BlockSpec((ROW_BLK, D_OUT), lambda i: (i, 0)),
        pl.BlockSpec((ROW_BLK, 1), lambda i: (i, 0)),
        pl.BlockSpec((1, D_OUT), lambda i: (0, 0)),
    ],
    out_specs=pl.BlockSpec((ROW_BLK, D_OUT), lambda i: (i, 0)),
    out_shape=jax.ShapeDtypeStruct((NP, D_OUT), jnp.float32),
)


def kernel(x, edge_index, W1, b1, W2, b2):
    ei = edge_index.astype(jnp.int32)
    pad_e = EP - E
    src = jnp.concatenate([ei[0], jnp.zeros((pad_e,), jnp.int32)])
    dst = jnp.concatenate([ei[1], jnp.full((pad_e,), N, jnp.int32)])
    srcm = src.reshape(EP // C, C)
    dstm = dst.reshape(EP // C, C)
    xp = jnp.pad(x, ((0, NP - N), (0, 0)))
    zeros2d = jnp.zeros((C, D_IN), jnp.float32)

    deg = _deg_call(dstm)
    s, xs = _scale_call(deg[0].reshape(NP, 1), deg[1].reshape(NP, 1), xp)
    agg1 = _agg_call(xs, srcm, dstm, zeros2d)
    gs = _layer_call(agg1[0], agg1[1], xs, s,
                     W1, b1.reshape(1, D_HID), W2)
    agg2 = _agg_call(gs, srcm, dstm, zeros2d)
    outp = _final_call(agg2[0], agg2[1], gs, s, b2.reshape(1, D_OUT))
    return outp[:N]
